# baseline (speedup 1.0000x reference)
import functools

import jax
import jax.numpy as jnp
from jax import lax
from jax.experimental import pallas as pl
from jax.experimental.pallas import tpu as pltpu
from jax.experimental.pallas import tpu_sc as plsc

VOCAB = 100000
EMBED = 20
TOK = 9
B = 1024
H = 500

NC, NS = 2, 16
NW = NC * NS
N_IDX = B * TOK
ROWS_PER_W = N_IDX // NW
CHUNK = 96
N_CHUNKS = ROWS_PER_W // CHUNK

VT = 3840
GRID = pl.cdiv(VOCAB, VT)


L = 16
OUT_PER_W = ROWS_PER_W * EMBED
ICHUNK = 128
N_ICHUNK = OUT_PER_W // ICHUNK
DMA_BATCH = 15


def _sc_gather(idx_hbm, table_hbm, out_hbm, idx_v, widx, out_v, sem):
    wid = lax.axis_index("s") * NC + lax.axis_index("c")
    base = wid * ROWS_PER_W

    pltpu.sync_copy(idx_hbm.at[pl.ds(base, ROWS_PER_W)], idx_v)

    def build_chunk(j, _):
        for s in range(ICHUNK // L):
            p = j * ICHUNK + s * L + lax.iota(jnp.int32, L)
            r = lax.div(p, EMBED)
            e = lax.rem(p, EMBED)
            wi = plsc.load_gather(idx_v, [r]) + e * VOCAB
            widx[j, pl.ds(s * L, L)] = wi
        pltpu.async_copy(table_hbm.at[widx.at[j]],
                         out_v.at[pl.ds(j * ICHUNK, ICHUNK)], sem)
        return ()

    lax.fori_loop(0, N_ICHUNK, build_chunk, ())
    pltpu.make_async_copy(out_hbm.at[pl.ds(wid * OUT_PER_W, OUT_PER_W)],
                          out_v, sem).wait()

    pltpu.sync_copy(out_v, out_hbm.at[pl.ds(wid * OUT_PER_W, OUT_PER_W)])


@functools.lru_cache(maxsize=1)
def _gather_call():
    return pl.kernel(
        _sc_gather,
        mesh=plsc.VectorSubcoreMesh(core_axis_name="c", subcore_axis_name="s"),
        out_type=jax.ShapeDtypeStruct((N_IDX * EMBED,), jnp.float32),
        scratch_types=[
            pltpu.VMEM((ROWS_PER_W,), jnp.int32),
            pltpu.VMEM((N_ICHUNK, ICHUNK), jnp.int32),
            pltpu.VMEM((OUT_PER_W,), jnp.float32),
            pltpu.SemaphoreType.DMA,
        ],
        compiler_params=pltpu.CompilerParams(needs_layout_passes=False),
    )


def _mlp_proj(h0_ref, W1_ref, b1_ref, W2_ref, b2_ref, W3_ref, b3_ref,
              outT_ref, h_scr):
    @pl.when(pl.program_id(0) == 0)
    def _():
        h = jnp.dot(h0_ref[...], W1_ref[...],
                    preferred_element_type=jnp.float32) + b1_ref[...]
        h = jnp.maximum(h, 0.0)
        for _ in range(6):
            h = jnp.dot(h, W2_ref[...],
                        preferred_element_type=jnp.float32) + b2_ref[...]
            h = jnp.maximum(h, 0.0)
        h_scr[...] = h.astype(jnp.bfloat16)

    outT = lax.dot_general(W3_ref[...].astype(jnp.bfloat16), h_scr[...],
                           (((0,), (1,)), ((), ())),
                           preferred_element_type=jnp.float32)
    ones = jnp.ones((1, B), dtype=jnp.float32)
    outT_ref[...] = outT + lax.dot_general(
        b3_ref[...], ones, (((0,), (0,)), ((), ())),
        preferred_element_type=jnp.float32)


def _tc_call(h0, W1, b1, W2, b2, W3, b3):
    return pl.pallas_call(
        _mlp_proj,
        grid=(GRID,),
        in_specs=[
            pl.BlockSpec((B, TOK * EMBED), lambda j: (0, 0)),
            pl.BlockSpec((TOK * EMBED, H), lambda j: (0, 0)),
            pl.BlockSpec((1, H), lambda j: (0, 0)),
            pl.BlockSpec((H, H), lambda j: (0, 0)),
            pl.BlockSpec((1, H), lambda j: (0, 0)),
            pl.BlockSpec((H, VT), lambda j: (0, j)),
            pl.BlockSpec((1, VT), lambda j: (0, j)),
        ],
        out_specs=pl.BlockSpec((VT, B), lambda j: (j, 0)),
        out_shape=jax.ShapeDtypeStruct((VOCAB, B), jnp.float32),
        scratch_shapes=[pltpu.VMEM((B, H), jnp.bfloat16)],
        compiler_params=pltpu.CompilerParams(
            dimension_semantics=("arbitrary",),
            vmem_limit_bytes=63 * 1024 * 1024),
    )(h0, W1, b1, W2, b2, W3, b3)


def kernel(x, emb, W1, b1, W2, b2, W3, b3):
    idx = x.reshape(-1).astype(jnp.int32)
    table = emb.T.reshape(-1)
    rows = _gather_call()(idx, table)
    h0 = rows.reshape(B, TOK * EMBED)
    outT = _tc_call(h0, W1, b1.reshape(1, H), W2, b2.reshape(1, H),
                    W3, b3.reshape(1, VOCAB))
    return outT.T


# speedup vs baseline: 1.0017x; 1.0017x over previous
import functools

import jax
import jax.numpy as jnp
from jax import lax
from jax.experimental import pallas as pl
from jax.experimental.pallas import tpu as pltpu
from jax.experimental.pallas import tpu_sc as plsc

VOCAB = 100000
EMBED = 20
TOK = 9
B = 1024
H = 500

NC, NS = 2, 16
NW = NC * NS
N_IDX = B * TOK
ROWS_PER_W = N_IDX // NW

VT = 3584
GRID = pl.cdiv(VOCAB, VT)


L = 16
OUT_PER_W = ROWS_PER_W * EMBED
ICHUNK = 128
N_ICHUNK = OUT_PER_W // ICHUNK


def _sc_gather(idx_hbm, table_hbm, out_hbm, idx_v, widx, out_v, sem):
    wid = lax.axis_index("s") * NC + lax.axis_index("c")
    base = wid * ROWS_PER_W

    pltpu.sync_copy(idx_hbm.at[pl.ds(base, ROWS_PER_W)], idx_v)

    def build_chunk(j, _):
        for s in range(ICHUNK // L):
            p = j * ICHUNK + s * L + lax.iota(jnp.int32, L)
            r = lax.div(p, EMBED)
            e = lax.rem(p, EMBED)
            wi = plsc.load_gather(idx_v, [r]) + e * VOCAB
            widx[j, pl.ds(s * L, L)] = wi
        pltpu.async_copy(table_hbm.at[widx.at[j]],
                         out_v.at[pl.ds(j * ICHUNK, ICHUNK)], sem)
        return ()

    lax.fori_loop(0, N_ICHUNK, build_chunk, ())
    pltpu.make_async_copy(out_hbm.at[pl.ds(wid * OUT_PER_W, OUT_PER_W)],
                          out_v, sem).wait()

    pltpu.sync_copy(out_v, out_hbm.at[pl.ds(wid * OUT_PER_W, OUT_PER_W)])


@functools.lru_cache(maxsize=1)
def _gather_call():
    return pl.kernel(
        _sc_gather,
        mesh=plsc.VectorSubcoreMesh(core_axis_name="c", subcore_axis_name="s"),
        out_type=jax.ShapeDtypeStruct((N_IDX * EMBED,), jnp.float32),
        scratch_types=[
            pltpu.VMEM((ROWS_PER_W,), jnp.int32),
            pltpu.VMEM((N_ICHUNK, ICHUNK), jnp.int32),
            pltpu.VMEM((OUT_PER_W,), jnp.float32),
            pltpu.SemaphoreType.DMA,
        ],
        compiler_params=pltpu.CompilerParams(needs_layout_passes=False),
    )


def _mlp_proj(h0_ref, W1_ref, b1_ref, W2_ref, b2_ref, W3_ref, b3_ref,
              outT_ref, h_scr):
    @pl.when(pl.program_id(0) == 0)
    def _():
        h = jnp.dot(h0_ref[...], W1_ref[...],
                    preferred_element_type=jnp.float32) + b1_ref[...]
        h = jnp.maximum(h, 0.0)
        for _ in range(6):
            h = jnp.dot(h, W2_ref[...],
                        preferred_element_type=jnp.float32) + b2_ref[...]
            h = jnp.maximum(h, 0.0)
        h_scr[...] = h.astype(jnp.bfloat16)

    outT = lax.dot_general(W3_ref[...].astype(jnp.bfloat16), h_scr[...],
                           (((0,), (1,)), ((), ())),
                           preferred_element_type=jnp.float32)
    ones = jnp.ones((1, B), dtype=jnp.float32)
    outT_ref[...] = outT + lax.dot_general(
        b3_ref[...], ones, (((0,), (0,)), ((), ())),
        preferred_element_type=jnp.float32)


def _tc_call(h0, W1, b1, W2, b2, W3, b3):
    return pl.pallas_call(
        _mlp_proj,
        grid=(GRID,),
        in_specs=[
            pl.BlockSpec((B, TOK * EMBED), lambda j: (0, 0)),
            pl.BlockSpec((TOK * EMBED, H), lambda j: (0, 0)),
            pl.BlockSpec((1, H), lambda j: (0, 0)),
            pl.BlockSpec((H, H), lambda j: (0, 0)),
            pl.BlockSpec((1, H), lambda j: (0, 0)),
            pl.BlockSpec((H, VT), lambda j: (0, j)),
            pl.BlockSpec((1, VT), lambda j: (0, j)),
        ],
        out_specs=pl.BlockSpec((VT, B), lambda j: (j, 0)),
        out_shape=jax.ShapeDtypeStruct((VOCAB, B), jnp.float32),
        scratch_shapes=[pltpu.VMEM((B, H), jnp.bfloat16)],
        compiler_params=pltpu.CompilerParams(
            dimension_semantics=("arbitrary",),
            vmem_limit_bytes=63 * 1024 * 1024),
    )(h0, W1, b1, W2, b2, W3, b3)


def kernel(x, emb, W1, b1, W2, b2, W3, b3):
    idx = x.reshape(-1).astype(jnp.int32)
    table = emb.T.reshape(-1)
    rows = _gather_call()(idx, table)
    h0 = rows.reshape(B, TOK * EMBED)
    outT = _tc_call(h0, W1, b1.reshape(1, H), W2, b2.reshape(1, H),
                    W3, b3.reshape(1, VOCAB))
    return outT.T
